# baseline (speedup 1.0000x reference)
import functools

import jax
import jax.numpy as jnp
from jax import lax
from jax.experimental import pallas as pl
from jax.experimental.pallas import tpu as pltpu
from jax.experimental.pallas import tpu_sc as plsc

B = 4096
V = 100000
E = 128
W = 50
NEG = 250
S = 304
NC = 2
NS = 16
NW = NC * NS
BPW = B // NW
SUB = 32
NSUB = BPW // SUB
NG = S // 16


NBUF = 4


def _sc_dots_body(center_hbm, idx_hbm, in_emb_hbm, out_emb_hbm, dots_hbm,
                  cidx_v, idx_v, crows_v, rows_0, rows_1, rows_2, rows_3,
                  part_v, dots_v, sem_0, sem_1, sem_2, sem_3, csem):
    cid = lax.axis_index("c")
    sid = lax.axis_index("s")
    wid = sid * NC + cid
    base = wid * BPW
    lanes = lax.iota(jnp.int32, 16)
    bufs = [rows_0, rows_1, rows_2, rows_3]
    sems = [sem_0, sem_1, sem_2, sem_3]

    def fire(bl, rows_buf, sem):
        o = bl * S
        pltpu.async_copy(out_emb_hbm.at[idx_v.at[pl.ds(o, 128)]],
                         rows_buf.at[pl.ds(0, 128)], sem)
        pltpu.async_copy(out_emb_hbm.at[idx_v.at[pl.ds(o + 128, 128)]],
                         rows_buf.at[pl.ds(128, 128)], sem)
        pltpu.async_copy(out_emb_hbm.at[idx_v.at[pl.ds(o + 256, S - 256)]],
                         rows_buf.at[pl.ds(256, S - 256)], sem)

    def drain(rows_buf, sem):
        pltpu.make_async_copy(out_emb_hbm.at[pl.ds(0, S)], rows_buf,
                              sem).wait()

    def compute(bl, rows_buf):
        c_chunks = [
            plsc.bitcast(crows_v[bl, pl.ds(k * 16, 16)], jnp.bfloat16)
            for k in range(E // 32)]

        def g_body(g, carry3):
            r0 = g * 16
            for l in range(16):
                r = r0 + l
                s = plsc.bitcast(rows_buf[r, pl.ds(0, 16)],
                                 jnp.bfloat16) * c_chunks[0]
                for k in range(1, E // 32):
                    s = s + plsc.bitcast(rows_buf[r, pl.ds(k * 16, 16)],
                                         jnp.bfloat16) * c_chunks[k]
                lo, hi = plsc.unpack(s, format=plsc.PackFormat.INTERLEAVED,
                                     preferred_element_type=jnp.float32)
                part_v[l, :] = lo + hi
            acc = plsc.load_gather(
                part_v, [lanes, jnp.zeros((16,), jnp.int32)])
            for c in range(1, 16):
                acc = acc + plsc.load_gather(
                    part_v, [lanes, jnp.full((16,), c, jnp.int32)])
            dots_v[pl.ds(bl * S + r0, 16)] = acc
            return carry3

        lax.fori_loop(0, NG, g_body, 0)

    def sub_body(subi, carry):
        sbase = base + subi * SUB
        pltpu.sync_copy(center_hbm.at[pl.ds(sbase, SUB)], cidx_v)
        pltpu.sync_copy(idx_hbm.at[pl.ds(sbase * S, SUB * S)], idx_v)
        pltpu.async_copy(in_emb_hbm.at[cidx_v], crows_v, csem).wait()

        for j in range(NBUF):
            fire(j, bufs[j], sems[j])

        def grp_body(bp, carry2):
            for j in range(NBUF):
                b = NBUF * bp + j
                drain(bufs[j], sems[j])
                compute(b, bufs[j])

                @pl.when(b + NBUF < SUB)
                def _():
                    fire(b + NBUF, bufs[j], sems[j])

            return carry2

        lax.fori_loop(0, SUB // NBUF, grp_body, 0)
        pltpu.sync_copy(dots_v, dots_hbm.at[pl.ds(sbase * S, SUB * S)])
        return carry

    lax.fori_loop(0, NSUB, sub_body, 0)


@functools.partial(jax.jit, static_argnames=("interpret",))
def _sc_dots(center, idx_flat, in_emb, out_emb, interpret=False):
    mesh = plsc.VectorSubcoreMesh(core_axis_name="c", subcore_axis_name="s",
                                  num_cores=NC, num_subcores=NS)
    return pl.kernel(
        _sc_dots_body,
        out_type=jax.ShapeDtypeStruct((B * S,), jnp.float32),
        mesh=mesh,
        scratch_types=[
            pltpu.VMEM((SUB,), jnp.int32),
            pltpu.VMEM((SUB * S,), jnp.int32),
            pltpu.VMEM((SUB, E // 2), jnp.int32),
            pltpu.VMEM((S, E // 2), jnp.int32),
            pltpu.VMEM((S, E // 2), jnp.int32),
            pltpu.VMEM((S, E // 2), jnp.int32),
            pltpu.VMEM((S, E // 2), jnp.int32),
            pltpu.VMEM((16, 16), jnp.float32),
            pltpu.VMEM((SUB * S,), jnp.float32),
            pltpu.SemaphoreType.DMA,
            pltpu.SemaphoreType.DMA,
            pltpu.SemaphoreType.DMA,
            pltpu.SemaphoreType.DMA,
            pltpu.SemaphoreType.DMA,
        ],
        compiler_params=pltpu.CompilerParams(needs_layout_passes=False,
                                             use_tc_tiling_on_sc=False),
        interpret=interpret,
    )(center, idx_flat, in_emb, out_emb)


BI = 256


def _tc_loss_body(dots_ref, pw_ref, out_ref, pos_s, neg_s):
    i = pl.program_id(0)

    @pl.when(i == 0)
    def _():
        d = dots_ref[...]
        pos = jax.nn.log_sigmoid(d[:, :W]) * pw_ref[...]
        neg = jax.nn.log_sigmoid(-d[:, W:W + NEG])
        pos_s[...] = jnp.sum(pos, axis=1)[None, :]
        neg_s[...] = jnp.sum(neg, axis=1)[None, :]

    nb = neg_s[0, pl.ds(i * BI, BI)]
    out_ref[...] = -(pos_s[...] + nb[:, None])


def _tc_loss(dots2d, pw, interpret=False):
    return pl.pallas_call(
        _tc_loss_body,
        grid=(B // BI,),
        in_specs=[
            pl.BlockSpec((B, S), lambda i: (0, 0)),
            pl.BlockSpec((B, W), lambda i: (0, 0)),
        ],
        out_specs=pl.BlockSpec((BI, B), lambda i: (i, 0)),
        out_shape=jax.ShapeDtypeStruct((B, B), jnp.float32),
        scratch_shapes=[
            pltpu.VMEM((1, B), jnp.float32),
            pltpu.VMEM((1, B), jnp.float32),
        ],
        interpret=interpret,
    )(dots2d, pw)


def kernel(center, positive, negative, in_emb, out_emb, *, _interpret=False):
    center = center.astype(jnp.int32)
    pos_idx = positive[:, :, 0].astype(jnp.int32)
    pw = positive[:, :, 1].astype(jnp.float32)
    neg_idx = negative.astype(jnp.int32)
    idx_flat = jnp.concatenate(
        [pos_idx, neg_idx, jnp.zeros((B, S - W - NEG), jnp.int32)],
        axis=1).reshape(-1)
    in_emb_i32 = jax.lax.bitcast_convert_type(
        in_emb.astype(jnp.bfloat16).reshape(V, E // 2, 2), jnp.int32)
    out_emb_i32 = jax.lax.bitcast_convert_type(
        out_emb.astype(jnp.bfloat16).reshape(V, E // 2, 2), jnp.int32)
    dots = _sc_dots(center, idx_flat, in_emb_i32, out_emb_i32,
                    interpret=_interpret)
    return _tc_loss(dots.reshape(B, S), pw, interpret=_interpret)


# speedup vs baseline: 1.8078x; 1.8078x over previous
import functools

import jax
import jax.numpy as jnp
from jax import lax
from jax.experimental import pallas as pl
from jax.experimental.pallas import tpu as pltpu
from jax.experimental.pallas import tpu_sc as plsc

B = 4096
V = 100000
E = 128
W = 50
NEG = 250
S = 304
NC = 2
NS = 16
NW = NC * NS
BPW = B // NW
SUB = 32
NSUB = BPW // SUB
NG = S // 16


NBUF = 4


def _sc_dots_body(center_hbm, idx_hbm, in_emb_hbm, out_emb_hbm, dots_hbm,
                  cidx_v, idx_v, crows_v, rows_0, rows_1, rows_2, rows_3,
                  part_v, dots_v, sem_0, sem_1, sem_2, sem_3, csem):
    cid = lax.axis_index("c")
    sid = lax.axis_index("s")
    wid = sid * NC + cid
    base = wid * BPW
    lanes = lax.iota(jnp.int32, 16)
    bufs = [rows_0, rows_1, rows_2, rows_3]
    sems = [sem_0, sem_1, sem_2, sem_3]

    def fire(bl, rows_buf, sem):
        o = bl * S
        pltpu.async_copy(out_emb_hbm.at[idx_v.at[pl.ds(o, 128)]],
                         rows_buf.at[pl.ds(0, 128)], sem)
        pltpu.async_copy(out_emb_hbm.at[idx_v.at[pl.ds(o + 128, 128)]],
                         rows_buf.at[pl.ds(128, 128)], sem)
        pltpu.async_copy(out_emb_hbm.at[idx_v.at[pl.ds(o + 256, S - 256)]],
                         rows_buf.at[pl.ds(256, S - 256)], sem)

    def drain(rows_buf, sem):
        pltpu.make_async_copy(out_emb_hbm.at[pl.ds(0, S)], rows_buf,
                              sem).wait()

    def compute(bl, rows_buf):
        c_chunks = [
            plsc.bitcast(crows_v[bl, pl.ds(k * 16, 16)], jnp.bfloat16)
            for k in range(E // 32)]

        def g_body(g, carry3):
            r0 = g * 16
            for l in range(16):
                r = r0 + l
                s = plsc.bitcast(rows_buf[r, pl.ds(0, 16)],
                                 jnp.bfloat16) * c_chunks[0]
                for k in range(1, E // 32):
                    s = s + plsc.bitcast(rows_buf[r, pl.ds(k * 16, 16)],
                                         jnp.bfloat16) * c_chunks[k]
                lo, hi = plsc.unpack(s, format=plsc.PackFormat.INTERLEAVED,
                                     preferred_element_type=jnp.float32)
                part_v[l, :] = lo + hi
            acc = plsc.load_gather(
                part_v, [lanes, jnp.zeros((16,), jnp.int32)])
            for c in range(1, 16):
                acc = acc + plsc.load_gather(
                    part_v, [lanes, jnp.full((16,), c, jnp.int32)])
            dots_v[pl.ds(bl * S + r0, 16)] = acc
            return carry3

        lax.fori_loop(0, NG, g_body, 0)

    def sub_body(subi, carry):
        sbase = base + subi * SUB
        pltpu.sync_copy(center_hbm.at[pl.ds(sbase, SUB)], cidx_v)
        pltpu.sync_copy(idx_hbm.at[pl.ds(sbase * S, SUB * S)], idx_v)
        pltpu.async_copy(in_emb_hbm.at[cidx_v], crows_v, csem).wait()

        for j in range(NBUF):
            fire(j, bufs[j], sems[j])

        def grp_body(bp, carry2):
            for j in range(NBUF):
                b = NBUF * bp + j
                drain(bufs[j], sems[j])
                compute(b, bufs[j])

                @pl.when(b + NBUF < SUB)
                def _():
                    fire(b + NBUF, bufs[j], sems[j])

            return carry2

        lax.fori_loop(0, SUB // NBUF, grp_body, 0)
        pltpu.sync_copy(dots_v, dots_hbm.at[pl.ds(sbase * S, SUB * S)])
        return carry

    lax.fori_loop(0, NSUB, sub_body, 0)


@functools.partial(jax.jit, static_argnames=("interpret",))
def _sc_dots(center, idx_flat, in_emb, out_emb, interpret=False):
    mesh = plsc.VectorSubcoreMesh(core_axis_name="c", subcore_axis_name="s",
                                  num_cores=NC, num_subcores=NS)
    return pl.kernel(
        _sc_dots_body,
        out_type=jax.ShapeDtypeStruct((B * S,), jnp.float32),
        mesh=mesh,
        scratch_types=[
            pltpu.VMEM((SUB,), jnp.int32),
            pltpu.VMEM((SUB * S,), jnp.int32),
            pltpu.VMEM((SUB, E // 2), jnp.int32),
            pltpu.VMEM((S, E // 2), jnp.int32),
            pltpu.VMEM((S, E // 2), jnp.int32),
            pltpu.VMEM((S, E // 2), jnp.int32),
            pltpu.VMEM((S, E // 2), jnp.int32),
            pltpu.VMEM((16, 16), jnp.float32),
            pltpu.VMEM((SUB * S,), jnp.float32),
            pltpu.SemaphoreType.DMA,
            pltpu.SemaphoreType.DMA,
            pltpu.SemaphoreType.DMA,
            pltpu.SemaphoreType.DMA,
            pltpu.SemaphoreType.DMA,
        ],
        compiler_params=pltpu.CompilerParams(needs_layout_passes=False,
                                             use_tc_tiling_on_sc=False),
        interpret=interpret,
    )(center, idx_flat, in_emb, out_emb)


BI = 256


def _tc_loss_body(dots_ref, pw_ref, out_ref, pos_s, neg_s):
    i = pl.program_id(0)

    @pl.when(i == 0)
    def _():
        d = dots_ref[...]
        pos = jax.nn.log_sigmoid(d[:, :W]) * pw_ref[...]
        neg = jax.nn.log_sigmoid(-d[:, W:W + NEG])
        pos_s[...] = jnp.sum(pos, axis=1)[None, :]
        neg_s[...] = jnp.sum(neg, axis=1)[None, :]

    nb = neg_s[0, pl.ds(i * BI, BI)]
    out_ref[...] = -(pos_s[...] + nb[:, None])


def _tc_loss(dots2d, pw, interpret=False):
    return pl.pallas_call(
        _tc_loss_body,
        grid=(B // BI,),
        in_specs=[
            pl.BlockSpec((B, S), lambda i: (0, 0)),
            pl.BlockSpec((B, W), lambda i: (0, 0)),
        ],
        out_specs=pl.BlockSpec((BI, B), lambda i: (i, 0)),
        out_shape=jax.ShapeDtypeStruct((B, B), jnp.float32),
        scratch_shapes=[
            pltpu.VMEM((1, B), jnp.float32),
            pltpu.VMEM((1, B), jnp.float32),
        ],
        interpret=interpret,
    )(dots2d, pw)


def _pack_bf16(table):
    u = jax.lax.bitcast_convert_type(table, jnp.uint32)
    r = (u + 0x7FFF + ((u >> 16) & 1)) >> 16
    return (r[:, :E // 2] | (r[:, E // 2:] << 16)).astype(jnp.int32)


def kernel(center, positive, negative, in_emb, out_emb, *, _interpret=False):
    center = center.astype(jnp.int32)
    pos_idx = positive[:, :, 0].astype(jnp.int32)
    pw = positive[:, :, 1].astype(jnp.float32)
    neg_idx = negative.astype(jnp.int32)
    idx_flat = jnp.concatenate(
        [pos_idx, neg_idx, jnp.zeros((B, S - W - NEG), jnp.int32)],
        axis=1).reshape(-1)
    dots = _sc_dots(center, idx_flat, _pack_bf16(in_emb),
                    _pack_bf16(out_emb), interpret=_interpret)
    return _tc_loss(dots.reshape(B, S), pw, interpret=_interpret)


# speedup vs baseline: 2.0862x; 1.1540x over previous
import functools

import jax
import jax.numpy as jnp
from jax import lax
from jax.experimental import pallas as pl
from jax.experimental.pallas import tpu as pltpu
from jax.experimental.pallas import tpu_sc as plsc

B = 4096
V = 100000
E = 128
W = 50
NEG = 250
S = 304
NC = 2
NS = 16
NW = NC * NS
BPW = B // NW
SUB = 32
NSUB = BPW // SUB
NG = S // 16


NBUF = 4


def _sc_dots_body(center_hbm, idx_hbm, in_emb_hbm, out_emb_hbm, dots_hbm,
                  cidx_v, idx_v, crowsf_v, crows_v, rows_0, rows_1, rows_2,
                  rows_3, part_v, dots_v, sem_0, sem_1, sem_2, sem_3, csem):
    cid = lax.axis_index("c")
    sid = lax.axis_index("s")
    wid = sid * NC + cid
    base = wid * BPW
    lanes = lax.iota(jnp.int32, 16)
    bufs = [rows_0, rows_1, rows_2, rows_3]
    sems = [sem_0, sem_1, sem_2, sem_3]

    def fire(bl, rows_buf, sem):
        o = bl * S
        pltpu.async_copy(out_emb_hbm.at[idx_v.at[pl.ds(o, 128)]],
                         rows_buf.at[pl.ds(0, 128)], sem)
        pltpu.async_copy(out_emb_hbm.at[idx_v.at[pl.ds(o + 128, 128)]],
                         rows_buf.at[pl.ds(128, 128)], sem)
        pltpu.async_copy(out_emb_hbm.at[idx_v.at[pl.ds(o + 256, S - 256)]],
                         rows_buf.at[pl.ds(256, S - 256)], sem)

    def drain(rows_buf, sem):
        pltpu.make_async_copy(out_emb_hbm.at[pl.ds(0, S)], rows_buf,
                              sem).wait()

    def compute(bl, rows_buf):
        c_chunks = [
            plsc.bitcast(crows_v[bl, pl.ds(k * 16, 16)], jnp.bfloat16)
            for k in range(E // 32)]

        def g_body(g, carry3):
            r0 = g * 16
            for l in range(16):
                r = r0 + l
                s = plsc.bitcast(rows_buf[r, pl.ds(0, 16)],
                                 jnp.bfloat16) * c_chunks[0]
                for k in range(1, E // 32):
                    s = s + plsc.bitcast(rows_buf[r, pl.ds(k * 16, 16)],
                                         jnp.bfloat16) * c_chunks[k]
                lo, hi = plsc.unpack(s, format=plsc.PackFormat.INTERLEAVED,
                                     preferred_element_type=jnp.float32)
                part_v[l, :] = lo + hi
            acc = plsc.load_gather(
                part_v, [lanes, jnp.zeros((16,), jnp.int32)])
            for c in range(1, 16):
                acc = acc + plsc.load_gather(
                    part_v, [lanes, jnp.full((16,), c, jnp.int32)])
            dots_v[pl.ds(bl * S + r0, 16)] = acc
            return carry3

        lax.fori_loop(0, NG, g_body, 0)

    def sub_body(subi, carry):
        sbase = base + subi * SUB
        pltpu.sync_copy(center_hbm.at[pl.ds(sbase, SUB)], cidx_v)
        pltpu.sync_copy(idx_hbm.at[pl.ds(sbase * S, SUB * S)], idx_v)
        pltpu.async_copy(in_emb_hbm.at[cidx_v], crowsf_v, csem).wait()

        def pack_body(bl, carryp):
            for k in range(E // 32):
                lo = plsc.bitcast(crowsf_v[bl, pl.ds(k * 16, 16)],
                                  jnp.uint32)
                hi = plsc.bitcast(
                    crowsf_v[bl, pl.ds(E // 2 + k * 16, 16)], jnp.uint32)
                lo = (lo + 0x7FFF + ((lo >> 16) & 1)) >> 16
                hi = (hi + 0x7FFF + ((hi >> 16) & 1)) >> 16
                crows_v[bl, pl.ds(k * 16, 16)] = plsc.bitcast(
                    lo | (hi << 16), jnp.int32)
            return carryp

        lax.fori_loop(0, SUB, pack_body, 0)

        for j in range(NBUF):
            fire(j, bufs[j], sems[j])

        def grp_body(bp, carry2):
            for j in range(NBUF):
                b = NBUF * bp + j
                drain(bufs[j], sems[j])
                compute(b, bufs[j])

                @pl.when(b + NBUF < SUB)
                def _():
                    fire(b + NBUF, bufs[j], sems[j])

            return carry2

        lax.fori_loop(0, SUB // NBUF, grp_body, 0)
        pltpu.sync_copy(dots_v, dots_hbm.at[pl.ds(sbase * S, SUB * S)])
        return carry

    lax.fori_loop(0, NSUB, sub_body, 0)


@functools.partial(jax.jit, static_argnames=("interpret",))
def _sc_dots(center, idx_flat, in_emb, out_emb, interpret=False):
    mesh = plsc.VectorSubcoreMesh(core_axis_name="c", subcore_axis_name="s",
                                  num_cores=NC, num_subcores=NS)
    return pl.kernel(
        _sc_dots_body,
        out_type=jax.ShapeDtypeStruct((B * S,), jnp.float32),
        mesh=mesh,
        scratch_types=[
            pltpu.VMEM((SUB,), jnp.int32),
            pltpu.VMEM((SUB * S,), jnp.int32),
            pltpu.VMEM((SUB, E), jnp.float32),
            pltpu.VMEM((SUB, E // 2), jnp.int32),
            pltpu.VMEM((S, E // 2), jnp.int32),
            pltpu.VMEM((S, E // 2), jnp.int32),
            pltpu.VMEM((S, E // 2), jnp.int32),
            pltpu.VMEM((S, E // 2), jnp.int32),
            pltpu.VMEM((16, 16), jnp.float32),
            pltpu.VMEM((SUB * S,), jnp.float32),
            pltpu.SemaphoreType.DMA,
            pltpu.SemaphoreType.DMA,
            pltpu.SemaphoreType.DMA,
            pltpu.SemaphoreType.DMA,
            pltpu.SemaphoreType.DMA,
        ],
        compiler_params=pltpu.CompilerParams(needs_layout_passes=False,
                                             use_tc_tiling_on_sc=False),
        interpret=interpret,
    )(center, idx_flat, in_emb, out_emb)


BI = 256


def _tc_loss_body(dots_ref, pw_ref, out_ref, pos_s, neg_s):
    i = pl.program_id(0)

    @pl.when(i == 0)
    def _():
        d = dots_ref[...]
        pos = jax.nn.log_sigmoid(d[:, :W]) * pw_ref[...]
        neg = jax.nn.log_sigmoid(-d[:, W:W + NEG])
        pos_s[...] = jnp.sum(pos, axis=1)[None, :]
        neg_s[...] = jnp.sum(neg, axis=1)[None, :]

    nb = neg_s[0, pl.ds(i * BI, BI)]
    out_ref[...] = -(pos_s[...] + nb[:, None])


def _tc_loss(dots2d, pw, interpret=False):
    return pl.pallas_call(
        _tc_loss_body,
        grid=(B // BI,),
        in_specs=[
            pl.BlockSpec((B, S), lambda i: (0, 0)),
            pl.BlockSpec((B, W), lambda i: (0, 0)),
        ],
        out_specs=pl.BlockSpec((BI, B), lambda i: (i, 0)),
        out_shape=jax.ShapeDtypeStruct((B, B), jnp.float32),
        scratch_shapes=[
            pltpu.VMEM((1, B), jnp.float32),
            pltpu.VMEM((1, B), jnp.float32),
        ],
        interpret=interpret,
    )(dots2d, pw)


def _pack_bf16(table):
    u = jax.lax.bitcast_convert_type(table, jnp.uint32)
    r = (u + 0x7FFF + ((u >> 16) & 1)) >> 16
    return (r[:, :E // 2] | (r[:, E // 2:] << 16)).astype(jnp.int32)


def kernel(center, positive, negative, in_emb, out_emb, *, _interpret=False):
    center = center.astype(jnp.int32)
    pos_idx = positive[:, :, 0].astype(jnp.int32)
    pw = positive[:, :, 1].astype(jnp.float32)
    neg_idx = negative.astype(jnp.int32)
    idx_flat = jnp.concatenate(
        [pos_idx, neg_idx, jnp.zeros((B, S - W - NEG), jnp.int32)],
        axis=1).reshape(-1)
    dots = _sc_dots(center, idx_flat, in_emb, _pack_bf16(out_emb),
                    interpret=_interpret)
    return _tc_loss(dots.reshape(B, S), pw, interpret=_interpret)
